# baseline (speedup 1.0000x reference)
import functools

import jax
import jax.numpy as jnp
from jax import lax
from jax.experimental import pallas as pl
from jax.experimental.pallas import tpu as pltpu
from jax.experimental.pallas import tpu_sc as plsc

N = 10000
E = 320000
D_IN = 128
D_HID = 256
D_OUT = 64
K = 10

NCORE = 2
NTILE = 16
HALF = D_OUT // NCORE
CHUNK = 128
NG = 3
GA = 2
GROUP = NG * GA
TCHUNKS = 162
EPT = TCHUNKS * CHUNK
LROWS = 640
ZROWS = LROWS
NPAD = NTILE * LROWS
SENT = N


def _mlp_body(x_ref, w1_ref, b1_ref, w2_ref, b2_ref, o_ref):
    h = jnp.maximum(
        jnp.dot(x_ref[...], w1_ref[...], preferred_element_type=jnp.float32)
        + b1_ref[...], 0.0)
    h2 = (jnp.dot(h, w2_ref[...], preferred_element_type=jnp.float32)
          + b2_ref[...])
    o_ref[0] = h2[:, :HALF]
    o_ref[1] = h2[:, HALF:]


def _mlp(x, W1, b1, W2, b2):
    R = 1000
    grid = N // R
    return pl.pallas_call(
        _mlp_body,
        grid=(grid,),
        in_specs=[
            pl.BlockSpec((R, D_IN), lambda i: (i, 0)),
            pl.BlockSpec((D_IN, D_HID), lambda i: (0, 0)),
            pl.BlockSpec((1, D_HID), lambda i: (0, 0)),
            pl.BlockSpec((D_HID, D_OUT), lambda i: (0, 0)),
            pl.BlockSpec((1, D_OUT), lambda i: (0, 0)),
        ],
        out_specs=pl.BlockSpec((NCORE, R, HALF), lambda i: (0, i, 0)),
        out_shape=jax.ShapeDtypeStruct((NCORE, N, HALF), jnp.float32),
    )(x, W1, b1.reshape(1, D_HID), W2, b2.reshape(1, D_OUT))


def _prop_body(h0, srcr, dstr, tempb, out,
               srcbuf, dstbuf, stage, zbuf, zerob, tbuf,
               ha, hb, sem, gsems, ssems):
    stage2 = stage.at[0]
    cid = lax.axis_index("c")
    tid = lax.axis_index("s")

    pltpu.sync_copy(srcr.at[tid], srcbuf)
    pltpu.sync_copy(dstr.at[tid], dstbuf)
    pltpu.sync_copy(tempb, tbuf)

    pltpu.sync_copy(h0.at[cid, pl.ds(tid * LROWS, LROWS)],
                    ha.at[pl.ds(tid * LROWS, LROWS)])

    pltpu.sync_copy(h0.at[cid, pl.ds(tid * ZROWS, ZROWS)], zbuf)
    t0 = tbuf[0, :]

    def _zscale(r, _):
        zbuf[r, pl.ds(0, 16)] = zbuf[r, pl.ds(0, 16)] * t0
        zbuf[r, pl.ds(16, 16)] = zbuf[r, pl.ds(16, 16)] * t0
        return 0
    lax.fori_loop(0, ZROWS, _zscale, 0)

    zv = jnp.zeros((16,), jnp.float32)

    def _zzero(r, _):
        zerob[r, pl.ds(0, 16)] = zv
        zerob[r, pl.ds(16, 16)] = zv
        return 0
    lax.fori_loop(0, 64, _zzero, 0)

    def _issue_zeros(buf):
        return [pltpu.async_copy(
            zerob, buf.at[pl.ds(tid * LROWS + zc * 64, 64)], sem)
            for zc in range(LROWS // 64)]

    zcps = _issue_zeros(hb)

    for i in range(K):
        cur, nxt = (ha, hb) if i % 2 == 0 else (hb, ha)

        for cp in zcps:
            cp.wait()
        plsc.subcore_barrier()

        def _gather(c, b, gsem):
            return pltpu.async_copy(
                cur.at[srcbuf.at[c]], stage.at[b], gsem)

        def _scatter(c, b, ssem):
            return pltpu.async_copy(
                stage.at[b], nxt.at[dstbuf.at[c]], ssem, add=True)

        gcps = [[_gather(g * GA + b, g * GA + b, gsems[g])
                 for b in range(GA)] for g in range(NG)]
        for g in range(NG):
            for b in range(GA):
                gcps[g][b].wait()
            for b in range(GA):
                _scatter(g * GA + b, g * GA + b, ssems[g])

        def _pipe(p, _):
            for g in range(NG):
                for b in range(GA):
                    s = g * GA + b
                    pltpu.make_async_copy(
                        stage.at[s],
                        nxt.at[dstbuf.at[NG * GA * p + s]],
                        ssems[g]).wait()
                for b in range(GA):
                    s = g * GA + b
                    _gather(NG * GA * p + s, s, gsems[g])
            for g in range(NG):
                for b in range(GA):
                    s = g * GA + b
                    pltpu.make_async_copy(
                        cur.at[srcbuf.at[NG * GA * p + s]],
                        stage.at[s], gsems[g]).wait()
                for b in range(GA):
                    s = g * GA + b
                    _scatter(NG * GA * p + s, s, ssems[g])
            return 0
        lax.fori_loop(1, TCHUNKS // (NG * GA), _pipe, 0)

        for g in range(NG):
            for b in range(GA):
                s = g * GA + b
                pltpu.make_async_copy(
                    stage.at[s], nxt.at[dstbuf.at[s]], ssems[g]).wait()
        plsc.subcore_barrier()

        if i < K - 1:
            zcps = _issue_zeros(cur)

        tv = tbuf[i + 1, :]
        for c5 in range(ZROWS // 128):
            pltpu.sync_copy(nxt.at[pl.ds(tid * ZROWS + c5 * 128, 128)],
                            stage2)

            def _zacc(r, _):
                row = c5 * 128 + r
                zbuf[row, pl.ds(0, 16)] = (
                    zbuf[row, pl.ds(0, 16)] + tv * stage2[r, pl.ds(0, 16)])
                zbuf[row, pl.ds(16, 16)] = (
                    zbuf[row, pl.ds(16, 16)] + tv * stage2[r, pl.ds(16, 16)])
                return 0
            lax.fori_loop(0, 128, _zacc, 0)

    pltpu.sync_copy(zbuf, out.at[cid, pl.ds(tid * ZROWS, ZROWS)])


def _propagate(h0p, srcr, dstr, tempb):
    mesh = plsc.VectorSubcoreMesh(core_axis_name="c", subcore_axis_name="s")
    return pl.kernel(
        _prop_body,
        out_type=jax.ShapeDtypeStruct((NCORE, NPAD, HALF), jnp.float32),
        mesh=mesh,
        compiler_params=pltpu.CompilerParams(use_tc_tiling_on_sc=False),
        scratch_types=[
            pltpu.VMEM((TCHUNKS, CHUNK), jnp.int32),
            pltpu.VMEM((TCHUNKS, CHUNK), jnp.int32),
            pltpu.VMEM((GROUP, CHUNK, HALF), jnp.float32),
            pltpu.VMEM((ZROWS, HALF), jnp.float32),
            pltpu.VMEM((64, HALF), jnp.float32),
            pltpu.VMEM((16, 16), jnp.float32),
            pltpu.VMEM_SHARED((NPAD, HALF), jnp.float32),
            pltpu.VMEM_SHARED((NPAD, HALF), jnp.float32),
            pltpu.SemaphoreType.DMA,
            [pltpu.SemaphoreType.DMA] * NG,
            [pltpu.SemaphoreType.DMA] * NG,
        ],
    )(h0p, srcr, dstr, tempb)


@jax.jit
def kernel(x, edge_index, W1, b1, W2, b2, temp):
    h0 = _mlp(x, W1, b1, W2, b2)
    h0p = jnp.pad(h0, ((0, 0), (0, NPAD - N), (0, 0)))

    dst = edge_index[0]
    src = edge_index[1]
    pad = NTILE * EPT - E
    srcp = jnp.pad(src, (0, pad), constant_values=SENT)
    dstp = jnp.pad(dst, (0, pad), constant_values=SENT)
    srcr = srcp.reshape(NTILE, TCHUNKS, CHUNK)
    dstr = dstp.reshape(NTILE, TCHUNKS, CHUNK)

    tpad = jnp.pad(temp, (0, 16 - (K + 1)))
    tempb = jnp.broadcast_to(tpad[:, None], (16, 16))

    z = _propagate(h0p, srcr, dstr, tempb)
    return z[:, :N].transpose(1, 0, 2).reshape(N, D_OUT)


# speedup vs baseline: 1.1884x; 1.1884x over previous
import functools

import jax
import jax.numpy as jnp
from jax import lax
from jax.experimental import pallas as pl
from jax.experimental.pallas import tpu as pltpu
from jax.experimental.pallas import tpu_sc as plsc

N = 10000
E = 320000
D_IN = 128
D_HID = 256
D_OUT = 64
K = 10

NCORE = 2
NTILE = 16
HALF = D_OUT // NCORE
CHUNK = 128
NG = 2
GA = 2
GROUP = NG * GA
TCHUNKS = 160
EPT = TCHUNKS * CHUNK
LROWS = 640
ZROWS = LROWS
NPAD = NTILE * LROWS
SENT = N


def _mlp_body(x_ref, w1_ref, b1_ref, w2_ref, b2_ref, o_ref):
    h = jnp.maximum(
        jnp.dot(x_ref[...], w1_ref[...], preferred_element_type=jnp.float32)
        + b1_ref[...], 0.0)
    h2 = (jnp.dot(h, w2_ref[...], preferred_element_type=jnp.float32)
          + b2_ref[...])
    o_ref[0] = h2[:, :HALF]
    o_ref[1] = h2[:, HALF:]


def _mlp(x, W1, b1, W2, b2):
    R = 1000
    grid = N // R
    return pl.pallas_call(
        _mlp_body,
        grid=(grid,),
        in_specs=[
            pl.BlockSpec((R, D_IN), lambda i: (i, 0)),
            pl.BlockSpec((D_IN, D_HID), lambda i: (0, 0)),
            pl.BlockSpec((1, D_HID), lambda i: (0, 0)),
            pl.BlockSpec((D_HID, D_OUT), lambda i: (0, 0)),
            pl.BlockSpec((1, D_OUT), lambda i: (0, 0)),
        ],
        out_specs=pl.BlockSpec((NCORE, R, HALF), lambda i: (0, i, 0)),
        out_shape=jax.ShapeDtypeStruct((NCORE, N, HALF), jnp.float32),
    )(x, W1, b1.reshape(1, D_HID), W2, b2.reshape(1, D_OUT))


def _prop_body(h0, srcr, dstr, tempb, out,
               srcbuf, dstbuf, stage, zbuf, zerob, tbuf,
               ha, hb, sem, gsems, ssems):
    stage2 = stage.at[0]
    cid = lax.axis_index("c")
    tid = lax.axis_index("s")

    pltpu.sync_copy(srcr.at[tid], srcbuf)
    pltpu.sync_copy(dstr.at[tid], dstbuf)
    pltpu.sync_copy(tempb, tbuf)

    pltpu.sync_copy(h0.at[cid, pl.ds(tid * LROWS, LROWS)],
                    ha.at[pl.ds(tid * LROWS, LROWS)])

    pltpu.sync_copy(h0.at[cid, pl.ds(tid * ZROWS, ZROWS)], zbuf)
    t0 = tbuf[0, :]

    def _zscale(r, _):
        zbuf[r, pl.ds(0, 16)] = zbuf[r, pl.ds(0, 16)] * t0
        zbuf[r, pl.ds(16, 16)] = zbuf[r, pl.ds(16, 16)] * t0
        return 0
    lax.fori_loop(0, ZROWS, _zscale, 0)

    zv = jnp.zeros((16,), jnp.float32)

    def _zzero(r, _):
        zerob[r, pl.ds(0, 16)] = zv
        zerob[r, pl.ds(16, 16)] = zv
        return 0
    lax.fori_loop(0, 64, _zzero, 0)

    def _issue_zeros(buf):
        return [pltpu.async_copy(
            zerob, buf.at[pl.ds(tid * LROWS + zc * 64, 64)], sem)
            for zc in range(LROWS // 64)]

    zcps = _issue_zeros(hb)

    for i in range(K):
        cur, nxt = (ha, hb) if i % 2 == 0 else (hb, ha)

        for cp in zcps:
            cp.wait()
        plsc.subcore_barrier()

        def _gather(c, b, gsem):
            return pltpu.async_copy(
                cur.at[srcbuf.at[c]], stage.at[b], gsem)

        def _scatter(c, b, ssem):
            return pltpu.async_copy(
                stage.at[b], nxt.at[dstbuf.at[c]], ssem, add=True)

        gcps = [[_gather(g * GA + b, g * GA + b, gsems[g])
                 for b in range(GA)] for g in range(NG)]
        for g in range(NG):
            for b in range(GA):
                gcps[g][b].wait()
            for b in range(GA):
                _scatter(g * GA + b, g * GA + b, ssems[g])

        def _pipe(p, _):
            for g in range(NG):
                for b in range(GA):
                    s = g * GA + b
                    pltpu.make_async_copy(
                        stage.at[s],
                        nxt.at[dstbuf.at[NG * GA * p + s]],
                        ssems[g]).wait()
                for b in range(GA):
                    s = g * GA + b
                    _gather(NG * GA * p + s, s, gsems[g])
            for g in range(NG):
                for b in range(GA):
                    s = g * GA + b
                    pltpu.make_async_copy(
                        cur.at[srcbuf.at[NG * GA * p + s]],
                        stage.at[s], gsems[g]).wait()
                for b in range(GA):
                    s = g * GA + b
                    _scatter(NG * GA * p + s, s, ssems[g])
            return 0
        lax.fori_loop(1, TCHUNKS // (NG * GA), _pipe, 0)

        for g in range(NG):
            for b in range(GA):
                s = g * GA + b
                pltpu.make_async_copy(
                    stage.at[s], nxt.at[dstbuf.at[s]], ssems[g]).wait()
        plsc.subcore_barrier()

        if i < K - 1:
            zcps = _issue_zeros(cur)

        tv = tbuf[i + 1, :]
        for c5 in range(ZROWS // 128):
            pltpu.sync_copy(nxt.at[pl.ds(tid * ZROWS + c5 * 128, 128)],
                            stage2)

            def _zacc(r, _):
                row = c5 * 128 + r
                zbuf[row, pl.ds(0, 16)] = (
                    zbuf[row, pl.ds(0, 16)] + tv * stage2[r, pl.ds(0, 16)])
                zbuf[row, pl.ds(16, 16)] = (
                    zbuf[row, pl.ds(16, 16)] + tv * stage2[r, pl.ds(16, 16)])
                return 0
            lax.fori_loop(0, 128, _zacc, 0)

    pltpu.sync_copy(zbuf, out.at[cid, pl.ds(tid * ZROWS, ZROWS)])


def _propagate(h0p, srcr, dstr, tempb):
    mesh = plsc.VectorSubcoreMesh(core_axis_name="c", subcore_axis_name="s")
    return pl.kernel(
        _prop_body,
        out_type=jax.ShapeDtypeStruct((NCORE, NPAD, HALF), jnp.float32),
        mesh=mesh,
        compiler_params=pltpu.CompilerParams(use_tc_tiling_on_sc=False),
        scratch_types=[
            pltpu.VMEM((TCHUNKS, CHUNK), jnp.int32),
            pltpu.VMEM((TCHUNKS, CHUNK), jnp.int32),
            pltpu.VMEM((GROUP, CHUNK, HALF), jnp.float32),
            pltpu.VMEM((ZROWS, HALF), jnp.float32),
            pltpu.VMEM((64, HALF), jnp.float32),
            pltpu.VMEM((16, 16), jnp.float32),
            pltpu.VMEM_SHARED((NPAD, HALF), jnp.float32),
            pltpu.VMEM_SHARED((NPAD, HALF), jnp.float32),
            pltpu.SemaphoreType.DMA,
            [pltpu.SemaphoreType.DMA] * NG,
            [pltpu.SemaphoreType.DMA] * NG,
        ],
    )(h0p, srcr, dstr, tempb)


@jax.jit
def kernel(x, edge_index, W1, b1, W2, b2, temp):
    h0 = _mlp(x, W1, b1, W2, b2)
    h0p = jnp.pad(h0, ((0, 0), (0, NPAD - N), (0, 0)))

    dst = edge_index[0]
    src = edge_index[1]
    pad = NTILE * EPT - E
    srcp = jnp.pad(src, (0, pad), constant_values=SENT)
    dstp = jnp.pad(dst, (0, pad), constant_values=SENT)
    srcr = srcp.reshape(NTILE, TCHUNKS, CHUNK)
    dstr = dstp.reshape(NTILE, TCHUNKS, CHUNK)

    tpad = jnp.pad(temp, (0, 16 - (K + 1)))
    tempb = jnp.broadcast_to(tpad[:, None], (16, 16))

    z = _propagate(h0p, srcr, dstr, tempb)
    return z[:, :N].transpose(1, 0, 2).reshape(N, D_OUT)
